# Initial kernel scaffold; baseline (speedup 1.0000x reference)
import jax
import jax.numpy as jnp
from jax.experimental import pallas as pl


def kernel(tempo, global_bar, global_pos, token, vel, dur, W_tempo, W_bar, W_pos, W_word, W_vel, W_dur, proj_W, proj_b):
    raise NotImplementedError("write your pallas kernel here")


# speedup vs baseline: 5.0661x; 5.0661x over previous
import functools

import jax
import jax.numpy as jnp
from jax import lax
from jax.experimental import pallas as pl
from jax.experimental.pallas import tpu as pltpu
from jax.experimental.pallas import tpu_sc as plsc

B, L, D = 4096, 20, 128
N = B * L
NF = 6
VPAD = 1024
NW = 32
CH = 64
PER_W = N // NW
NIT = PER_W // CH
IDXROWS = NF * CH // 128


def _fuse_body(w_ref, p_ref, b_ref, o_ref):
    t = lax.dot_general(
        w_ref[0], p_ref[0], (((1,), (1,)), ((), ())),
        preferred_element_type=jnp.float32,
    )
    scale = jnp.where(pl.program_id(0) == 0, 1.0, 0.0)
    o_ref[0] = t + scale * b_ref[...]


_fuse = pl.pallas_call(
    _fuse_body,
    grid=(NF,),
    in_specs=[
        pl.BlockSpec((1, VPAD, D), lambda f: (f, 0, 0)),
        pl.BlockSpec((1, D, D), lambda f: (f, 0, 0)),
        pl.BlockSpec((1, D), lambda f: (0, 0)),
    ],
    out_specs=pl.BlockSpec((1, VPAD, D), lambda f: (f, 0, 0)),
    out_shape=jax.ShapeDtypeStruct((NF, VPAD, D), jnp.float32),
)


def _sc_body(idx_hbm, tab_hbm, out_hbm, idxv, gv, ov, sem):
    wid = lax.axis_index("s") * 2 + lax.axis_index("c")

    def chunk(t, carry):
        pltpu.sync_copy(idx_hbm.at[pl.ds((wid * NIT + t) * (NF * CH), NF * CH)], idxv)
        for p in range(NF * CH // 16):
            f = (p * 16) // CH
            sl = pl.ds(p * 16, 16)
            idxv[sl] = idxv[sl] + jnp.int32(f * VPAD)
        cps = [
            pltpu.async_copy(tab_hbm.at[idxv.at[pl.ds(r * 128, 128)]],
                             gv.at[pl.ds(r * 128, 128)], sem)
            for r in range(IDXROWS)
        ]
        for cp in cps:
            cp.wait()

        def tok(j, carry2):
            for c in range(8):
                sl = pl.ds(c * 16, 16)
                acc = gv[j, sl]
                for f in range(1, NF):
                    acc = acc + gv[f * CH + j, sl]
                ov[j, sl] = acc
            return carry2

        lax.fori_loop(0, CH, tok, 0, unroll=2)
        pltpu.sync_copy(ov, out_hbm.at[pl.ds(wid * PER_W + t * CH, CH)])
        return carry

    lax.fori_loop(0, NIT, chunk, 0)


_sc_gather = functools.partial(
    pl.kernel,
    out_type=jax.ShapeDtypeStruct((N, D), jnp.float32),
    mesh=plsc.VectorSubcoreMesh(core_axis_name="c", subcore_axis_name="s"),
    scratch_types=[
        pltpu.VMEM((NF * CH,), jnp.int32),
        pltpu.VMEM((NF * CH, D), jnp.float32),
        pltpu.VMEM((CH, D), jnp.float32),
        pltpu.SemaphoreType.DMA,
    ],
)(_sc_body)


@jax.jit
def kernel(tempo, global_bar, global_pos, token, vel, dur,
           W_tempo, W_bar, W_pos, W_word, W_vel, W_dur, proj_W, proj_b):
    tables = [W_tempo, W_bar, W_pos, W_word, W_vel, W_dur]
    w_stack = jnp.stack(
        [jnp.pad(w, ((0, VPAD - w.shape[0]), (0, 0))) for w in tables]
    )
    p_stack = proj_W.reshape(D, NF, D).transpose(1, 0, 2)
    fused = _fuse(w_stack, p_stack, proj_b.reshape(1, D)).reshape(NF * VPAD, D)

    idx6 = jnp.stack([tempo, global_bar, global_pos, token, vel, dur])
    idx = (idx6.astype(jnp.int32)
           .reshape(NF, NW, NIT, CH)
           .transpose(1, 2, 0, 3)
           .reshape(NW * NIT * NF * CH))

    out = _sc_gather(idx, fused)
    return out.reshape(B, L, D)


# speedup vs baseline: 5.2395x; 1.0342x over previous
import functools

import jax
import jax.numpy as jnp
from jax import lax
from jax.experimental import pallas as pl
from jax.experimental.pallas import tpu as pltpu
from jax.experimental.pallas import tpu_sc as plsc

B, L, D = 4096, 20, 128
N = B * L
NF = 6
VPAD = 1024
NW = 32
CH = 64
PER_W = N // NW
NIT = PER_W // CH
NSUP = NIT // 2


def _fuse_body(w_ref, p_ref, b_ref, o_ref):
    t = lax.dot_general(
        w_ref[0], p_ref[0], (((1,), (1,)), ((), ())),
        preferred_element_type=jnp.float32,
    )
    scale = jnp.where(pl.program_id(0) == 0, 1.0, 0.0)
    o_ref[0] = t + scale * b_ref[...]


_fuse = pl.pallas_call(
    _fuse_body,
    grid=(NF,),
    in_specs=[
        pl.BlockSpec((1, VPAD, D), lambda f: (f, 0, 0)),
        pl.BlockSpec((1, D, D), lambda f: (f, 0, 0)),
        pl.BlockSpec((1, D), lambda f: (0, 0)),
    ],
    out_specs=pl.BlockSpec((1, VPAD, D), lambda f: (f, 0, 0)),
    out_shape=jax.ShapeDtypeStruct((NF, VPAD, D), jnp.float32),
)


def _sc_body(i0, i1, i2, i3, i4, i5, tab_hbm, out_hbm,
             idxv, gv0, gv1, ov0, ov1, semg, semo):
    wid = lax.axis_index("s") * 2 + lax.axis_index("c")
    base = wid * PER_W

    cps = [
        pltpu.async_copy(idx.at[pl.ds(base, PER_W)],
                         idxv.at[pl.ds(f * PER_W, PER_W)], semg)
        for f, idx in enumerate((i0, i1, i2, i3, i4, i5))
    ]
    for cp in cps:
        cp.wait()

    for f in range(1, NF):
        def off_body(p, carry, f=f):
            sl = pl.ds(f * PER_W + p * 16, 16)
            idxv[sl] = idxv[sl] + jnp.int32(f * VPAD)
            return carry
        lax.fori_loop(0, PER_W // 16, off_body, 0, unroll=4)

    def fire(t, gvb):
        for f in range(NF):
            pltpu.async_copy(
                tab_hbm.at[idxv.at[pl.ds(f * PER_W + t * CH, CH)]],
                gvb.at[pl.ds(f * CH, CH)], semg)

    def drain_gathers(gvb):
        pltpu.make_async_copy(tab_hbm.at[pl.ds(0, NF * CH)], gvb, semg).wait()

    def drain_out(ovb):
        pltpu.make_async_copy(ovb, out_hbm.at[pl.ds(base, CH)], semo).wait()

    def compute(gvb, ovb):
        def tok(j, carry):
            for c in range(8):
                sl = pl.ds(c * 16, 16)
                acc = gvb[j, sl]
                for f in range(1, NF):
                    acc = acc + gvb[f * CH + j, sl]
                ovb[j, sl] = acc
            return carry
        lax.fori_loop(0, CH, tok, 0, unroll=2)

    fire(0, gv0)
    fire(1, gv1)

    def sup(s, carry):
        for half, (gvb, ovb) in enumerate(((gv0, ov0), (gv1, ov1))):
            t = 2 * s + half
            drain_gathers(gvb)

            @pl.when(s > 0)
            def _():
                drain_out(ovb)

            compute(gvb, ovb)
            pltpu.async_copy(ovb, out_hbm.at[pl.ds(base + t * CH, CH)], semo)

            @pl.when(s < NSUP - 1)
            def _():
                fire(t + 2, gvb)
        return carry

    lax.fori_loop(0, NSUP, sup, 0)
    drain_out(ov0)
    drain_out(ov1)


_sc_gather = functools.partial(
    pl.kernel,
    out_type=jax.ShapeDtypeStruct((N, D), jnp.float32),
    mesh=plsc.VectorSubcoreMesh(core_axis_name="c", subcore_axis_name="s"),
    scratch_types=[
        pltpu.VMEM((NF * PER_W,), jnp.int32),
        pltpu.VMEM((NF * CH, D), jnp.float32),
        pltpu.VMEM((NF * CH, D), jnp.float32),
        pltpu.VMEM((CH, D), jnp.float32),
        pltpu.VMEM((CH, D), jnp.float32),
        pltpu.SemaphoreType.DMA,
        pltpu.SemaphoreType.DMA,
    ],
)(_sc_body)


@jax.jit
def kernel(tempo, global_bar, global_pos, token, vel, dur,
           W_tempo, W_bar, W_pos, W_word, W_vel, W_dur, proj_W, proj_b):
    tables = [W_tempo, W_bar, W_pos, W_word, W_vel, W_dur]
    w_stack = jnp.stack(
        [jnp.pad(w, ((0, VPAD - w.shape[0]), (0, 0))) for w in tables]
    )
    p_stack = proj_W.reshape(D, NF, D).transpose(1, 0, 2)
    fused = _fuse(w_stack, p_stack, proj_b.reshape(1, D)).reshape(NF * VPAD, D)

    idxs = [a.astype(jnp.int32).reshape(N)
            for a in (tempo, global_bar, global_pos, token, vel, dur)]
    out = _sc_gather(*idxs, fused)
    return out.reshape(B, L, D)


# speedup vs baseline: 6.7947x; 1.2968x over previous
import functools

import jax
import jax.numpy as jnp
from jax import lax
from jax.experimental import pallas as pl
from jax.experimental.pallas import tpu as pltpu
from jax.experimental.pallas import tpu_sc as plsc

B, L, D = 4096, 20, 128
N = B * L
NF = 6
NG = 4
VPAD = 1024
NW = 32
CH = 64
PER_W = N // NW
NIT = PER_W // CH
NSUP = NIT // 2

R_TV, R_BAR, R_PD, R_WORD = 0, 4096, 4096 + 256, 4096 + 256 + 16384
TAB_ROWS = R_WORD + 1024
TV_BLKS, BAR_BLKS, PD_BLKS, WORD_BLKS = 32, 2, 128, 8


def _fuse_body(w_ref, p_ref, o_ref):
    o_ref[0] = lax.dot_general(
        w_ref[0], p_ref[0], (((1,), (1,)), ((), ())),
        preferred_element_type=jnp.float32,
    )


_fuse = pl.pallas_call(
    _fuse_body,
    grid=(NF,),
    in_specs=[
        pl.BlockSpec((1, VPAD, D), lambda f: (f, 0, 0)),
        pl.BlockSpec((1, D, D), lambda f: (f, 0, 0)),
    ],
    out_specs=pl.BlockSpec((1, VPAD, D), lambda f: (f, 0, 0)),
    out_shape=jax.ShapeDtypeStruct((NF, VPAD, D), jnp.float32),
)


def _combine_body(t_ref, b_ref, o_ref):
    p = pl.program_id(0)

    @pl.when(p < TV_BLKS)
    def _():
        tvel = t_ref[4, 0:CH, :] + b_ref[...]
        for h in range(2):
            row = t_ref[0, 2 * p + h, :]
            o_ref[pl.ds(h * CH, CH)] = tvel + row[None, :]

    @pl.when((p >= TV_BLKS) & (p < TV_BLKS + BAR_BLKS))
    def _():
        o_ref[...] = t_ref[1, pl.ds((p - TV_BLKS) * 128, 128), :]

    @pl.when((p >= TV_BLKS + BAR_BLKS) & (p < TV_BLKS + BAR_BLKS + PD_BLKS))
    def _():
        row = t_ref[2, p - (TV_BLKS + BAR_BLKS), :]
        o_ref[...] = t_ref[5, 0:128, :] + row[None, :]

    @pl.when(p >= TV_BLKS + BAR_BLKS + PD_BLKS)
    def _():
        o_ref[...] = t_ref[3, pl.ds((p - (TV_BLKS + BAR_BLKS + PD_BLKS)) * 128, 128), :]


_combine = pl.pallas_call(
    _combine_body,
    grid=(TAB_ROWS // 128,),
    in_specs=[
        pl.BlockSpec((NF, VPAD, D), lambda p: (0, 0, 0)),
        pl.BlockSpec((1, D), lambda p: (0, 0)),
    ],
    out_specs=pl.BlockSpec((128, D), lambda p: (p, 0)),
    out_shape=jax.ShapeDtypeStruct((TAB_ROWS, D), jnp.float32),
)


def _sc_body(i0, i1, i2, i3, i4, i5, tab_hbm, out_hbm,
             idxv, gv0, gv1, ov0, ov1, semg, semo):
    wid = lax.axis_index("s") * 2 + lax.axis_index("c")
    base = wid * PER_W

    cps = [
        pltpu.async_copy(idx.at[pl.ds(base, PER_W)],
                         idxv.at[pl.ds(f * PER_W, PER_W)], semg)
        for f, idx in enumerate((i0, i1, i2, i3, i4, i5))
    ]
    for cp in cps:
        cp.wait()

    def comb_body(p, carry):
        s0 = pl.ds(0 * PER_W + p * 16, 16)
        s1 = pl.ds(1 * PER_W + p * 16, 16)
        s2 = pl.ds(2 * PER_W + p * 16, 16)
        s3 = pl.ds(3 * PER_W + p * 16, 16)
        s4 = pl.ds(4 * PER_W + p * 16, 16)
        s5 = pl.ds(5 * PER_W + p * 16, 16)
        idxv[s0] = idxv[s0] * 64 + idxv[s4]
        idxv[s1] = idxv[s1] + jnp.int32(R_BAR)
        idxv[s2] = idxv[s2] * 128 + idxv[s5] + jnp.int32(R_PD)
        idxv[s3] = idxv[s3] + jnp.int32(R_WORD)
        return carry

    lax.fori_loop(0, PER_W // 16, comb_body, 0, unroll=4)

    def fire(t, gvb):
        for g in range(NG):
            pltpu.async_copy(
                tab_hbm.at[idxv.at[pl.ds(g * PER_W + t * CH, CH)]],
                gvb.at[pl.ds(g * CH, CH)], semg)

    def drain_gathers(gvb):
        pltpu.make_async_copy(tab_hbm.at[pl.ds(0, NG * CH)], gvb, semg).wait()

    def drain_out(ovb):
        pltpu.make_async_copy(ovb, out_hbm.at[pl.ds(base, CH)], semo).wait()

    def compute(gvb, ovb):
        def tok(j, carry):
            for c in range(8):
                sl = pl.ds(c * 16, 16)
                acc = gvb[j, sl]
                for g in range(1, NG):
                    acc = acc + gvb[g * CH + j, sl]
                ovb[j, sl] = acc
            return carry
        lax.fori_loop(0, CH, tok, 0, unroll=2)

    fire(0, gv0)
    fire(1, gv1)

    def sup(s, carry):
        for half, (gvb, ovb) in enumerate(((gv0, ov0), (gv1, ov1))):
            t = 2 * s + half
            drain_gathers(gvb)

            @pl.when(s > 0)
            def _():
                drain_out(ovb)

            compute(gvb, ovb)
            pltpu.async_copy(ovb, out_hbm.at[pl.ds(base + t * CH, CH)], semo)

            @pl.when(s < NSUP - 1)
            def _():
                fire(t + 2, gvb)
        return carry

    lax.fori_loop(0, NSUP, sup, 0)
    drain_out(ov0)
    drain_out(ov1)


_sc_gather = functools.partial(
    pl.kernel,
    out_type=jax.ShapeDtypeStruct((N, D), jnp.float32),
    mesh=plsc.VectorSubcoreMesh(core_axis_name="c", subcore_axis_name="s"),
    scratch_types=[
        pltpu.VMEM((NF * PER_W,), jnp.int32),
        pltpu.VMEM((NG * CH, D), jnp.float32),
        pltpu.VMEM((NG * CH, D), jnp.float32),
        pltpu.VMEM((CH, D), jnp.float32),
        pltpu.VMEM((CH, D), jnp.float32),
        pltpu.SemaphoreType.DMA,
        pltpu.SemaphoreType.DMA,
    ],
)(_sc_body)


@jax.jit
def kernel(tempo, global_bar, global_pos, token, vel, dur,
           W_tempo, W_bar, W_pos, W_word, W_vel, W_dur, proj_W, proj_b):
    tables = [W_tempo, W_bar, W_pos, W_word, W_vel, W_dur]
    w_stack = jnp.stack(
        [jnp.pad(w, ((0, VPAD - w.shape[0]), (0, 0))) for w in tables]
    )
    p_stack = proj_W.reshape(D, NF, D).transpose(1, 0, 2)
    fused = _fuse(w_stack, p_stack)
    tab = _combine(fused, proj_b.reshape(1, D))

    idxs = [a.astype(jnp.int32).reshape(N)
            for a in (tempo, global_bar, global_pos, token, vel, dur)]
    out = _sc_gather(*idxs, tab)
    return out.reshape(B, L, D)


# speedup vs baseline: 6.8152x; 1.0030x over previous
import functools

import jax
import jax.numpy as jnp
from jax import lax
from jax.experimental import pallas as pl
from jax.experimental.pallas import tpu as pltpu
from jax.experimental.pallas import tpu_sc as plsc

B, L, D = 4096, 20, 128
N = B * L
NF = 6
NG = 4
VPAD = 1024
NW = 32
CH = 64
PER_W = N // NW
NIT = PER_W // CH
NSUP = NIT // 2

R_TV, R_BAR, R_PD, R_WORD = 0, 4096, 4096 + 256, 4096 + 256 + 16384
TAB_ROWS = R_WORD + 1024
TV_BLKS, BAR_BLKS, PD_BLKS, WORD_BLKS = 32, 2, 128, 8


def _fuse_body(w_ref, p_ref, o_ref):
    o_ref[0] = lax.dot_general(
        w_ref[0], p_ref[0], (((1,), (1,)), ((), ())),
        preferred_element_type=jnp.float32,
    )


_fuse = pl.pallas_call(
    _fuse_body,
    grid=(NF,),
    in_specs=[
        pl.BlockSpec((1, VPAD, D), lambda f: (f, 0, 0)),
        pl.BlockSpec((1, D, D), lambda f: (f, 0, 0)),
    ],
    out_specs=pl.BlockSpec((1, VPAD, D), lambda f: (f, 0, 0)),
    out_shape=jax.ShapeDtypeStruct((NF, VPAD, D), jnp.float32),
)


def _combine_body(t_ref, b_ref, o_ref):
    p = pl.program_id(0)

    @pl.when(p < TV_BLKS)
    def _():
        tvel = t_ref[4, 0:CH, :] + b_ref[...]
        for h in range(2):
            row = t_ref[0, 2 * p + h, :]
            o_ref[pl.ds(h * CH, CH)] = tvel + row[None, :]

    @pl.when((p >= TV_BLKS) & (p < TV_BLKS + BAR_BLKS))
    def _():
        o_ref[...] = t_ref[1, pl.ds((p - TV_BLKS) * 128, 128), :]

    @pl.when((p >= TV_BLKS + BAR_BLKS) & (p < TV_BLKS + BAR_BLKS + PD_BLKS))
    def _():
        row = t_ref[2, p - (TV_BLKS + BAR_BLKS), :]
        o_ref[...] = t_ref[5, 0:128, :] + row[None, :]

    @pl.when(p >= TV_BLKS + BAR_BLKS + PD_BLKS)
    def _():
        o_ref[...] = t_ref[3, pl.ds((p - (TV_BLKS + BAR_BLKS + PD_BLKS)) * 128, 128), :]


_combine = pl.pallas_call(
    _combine_body,
    grid=(TAB_ROWS // 128,),
    in_specs=[
        pl.BlockSpec((NF, VPAD, D), lambda p: (0, 0, 0)),
        pl.BlockSpec((1, D), lambda p: (0, 0)),
    ],
    out_specs=pl.BlockSpec((128, D), lambda p: (p, 0)),
    out_shape=jax.ShapeDtypeStruct((TAB_ROWS, D), jnp.float32),
)


def _sc_body(i0, i1, i2, i3, i4, i5, tab_hbm, out_hbm,
             idxv, gv0, gv1, ov0, ov1, semg, semo):
    wid = lax.axis_index("s") * 2 + lax.axis_index("c")
    base = wid * PER_W

    cps = [
        pltpu.async_copy(idx.at[pl.ds(base, PER_W)],
                         idxv.at[pl.ds(f * PER_W, PER_W)], semg)
        for f, idx in enumerate((i0, i1, i2, i3, i4, i5))
    ]
    for cp in cps:
        cp.wait()

    def comb_body(p, carry):
        s0 = pl.ds(0 * PER_W + p * 16, 16)
        s1 = pl.ds(1 * PER_W + p * 16, 16)
        s2 = pl.ds(2 * PER_W + p * 16, 16)
        s3 = pl.ds(3 * PER_W + p * 16, 16)
        s4 = pl.ds(4 * PER_W + p * 16, 16)
        s5 = pl.ds(5 * PER_W + p * 16, 16)
        idxv[s0] = idxv[s0] * 64 + idxv[s4]
        idxv[s1] = idxv[s1] + jnp.int32(R_BAR)
        idxv[s2] = idxv[s2] * 128 + idxv[s5] + jnp.int32(R_PD)
        idxv[s3] = idxv[s3] + jnp.int32(R_WORD)
        return carry

    lax.fori_loop(0, PER_W // 16, comb_body, 0, unroll=4)

    def fire(t, gvb):
        for g in range(NG):
            pltpu.async_copy(
                tab_hbm.at[idxv.at[pl.ds(g * PER_W + t * CH, CH)]],
                gvb.at[pl.ds(g * CH, CH)], semg)

    def drain_gathers(gvb):
        pltpu.make_async_copy(tab_hbm.at[pl.ds(0, NG * CH)], gvb, semg).wait()

    def drain_out(ovb):
        pltpu.make_async_copy(ovb, out_hbm.at[pl.ds(base, CH)], semo).wait()

    def compute(gvb, ovb):
        def tok(j, carry):
            for c in range(8):
                sl = pl.ds(c * 16, 16)
                acc = gvb[j, sl]
                for g in range(1, NG):
                    acc = acc + gvb[g * CH + j, sl]
                ovb[j, sl] = acc
            return carry
        lax.fori_loop(0, CH, tok, 0, unroll=2)

    fire(0, gv0)
    fire(1, gv1)

    def sup(s, carry):
        for half, (gvb, ovb) in enumerate(((gv0, ov0), (gv1, ov1))):
            t = 2 * s + half
            drain_gathers(gvb)

            @pl.when(s > 0)
            def _():
                drain_out(ovb)

            compute(gvb, ovb)
            pltpu.async_copy(ovb, out_hbm.at[pl.ds(base + t * CH, CH)], semo)

            @pl.when(s < NSUP - 1)
            def _():
                fire(t + 2, gvb)
        return carry

    lax.fori_loop(0, NSUP, sup, 0)
    drain_out(ov0)
    drain_out(ov1)


_sc_gather = functools.partial(
    pl.kernel,
    out_type=jax.ShapeDtypeStruct((N, D), jnp.float32),
    mesh=plsc.VectorSubcoreMesh(core_axis_name="c", subcore_axis_name="s",
                                num_cores=2),
    scratch_types=[
        pltpu.VMEM((NF * PER_W,), jnp.int32),
        pltpu.VMEM((NG * CH, D), jnp.float32),
        pltpu.VMEM((NG * CH, D), jnp.float32),
        pltpu.VMEM((CH, D), jnp.float32),
        pltpu.VMEM((CH, D), jnp.float32),
        pltpu.SemaphoreType.DMA,
        pltpu.SemaphoreType.DMA,
    ],
)(_sc_body)


@jax.jit
def kernel(tempo, global_bar, global_pos, token, vel, dur,
           W_tempo, W_bar, W_pos, W_word, W_vel, W_dur, proj_W, proj_b):
    tables = [W_tempo, W_bar, W_pos, W_word, W_vel, W_dur]
    w_stack = jnp.stack(
        [jnp.pad(w, ((0, VPAD - w.shape[0]), (0, 0))) for w in tables]
    )
    p_stack = proj_W.reshape(D, NF, D).transpose(1, 0, 2)
    fused = _fuse(w_stack, p_stack)
    tab = _combine(fused, proj_b.reshape(1, D))

    idxs = [a.astype(jnp.int32).reshape(N)
            for a in (tempo, global_bar, global_pos, token, vel, dur)]
    out = _sc_gather(*idxs, tab)
    return out.reshape(B, L, D)


# speedup vs baseline: 8.0869x; 1.1866x over previous
import functools

import jax
import jax.numpy as jnp
from jax import lax
from jax.experimental import pallas as pl
from jax.experimental.pallas import tpu as pltpu
from jax.experimental.pallas import tpu_sc as plsc

B, L, D = 4096, 20, 128
N = B * L
NF = 6
NG = 4
VPAD = 1024
NW = 32
CH = 64
PER_W = N // NW
NIT = PER_W // CH
NSUP = NIT // 2

R_TV, R_BAR, R_PD, R_WORD = 0, 4096, 4096 + 256, 4096 + 256 + 16384
TAB_ROWS = R_WORD + 1024
TV_BLKS, BAR_BLKS, PD_BLKS, WORD_BLKS = 32, 2, 128, 8


def _build_body(w0, w1, w2, w3, w4, w5, p_ref, b_ref, o_ref):
    def fuse(w, f):
        return lax.dot_general(
            w[...], p_ref[:, pl.ds(f * D, D)], (((1,), (1,)), ((), ())),
            preferred_element_type=jnp.float32,
        )

    tempo, bar, pos = fuse(w0, 0), fuse(w1, 1), fuse(w2, 2)
    word, vel, dur = fuse(w3, 3), fuse(w4, 4), fuse(w5, 5)
    tv = tempo[:, None, :] + (vel + b_ref[...])[None, :, :]
    o_ref[pl.ds(R_TV, R_BAR)] = tv.reshape(R_BAR, D)
    o_ref[pl.ds(R_BAR, 256)] = bar
    pd = pos[:, None, :] + dur[None, :, :]
    o_ref[pl.ds(R_PD, R_WORD - R_PD)] = pd.reshape(R_WORD - R_PD, D)
    o_ref[pl.ds(R_WORD, 1024)] = word


_build = pl.pallas_call(
    _build_body,
    out_shape=jax.ShapeDtypeStruct((TAB_ROWS, D), jnp.float32),
)


def _sc_body(i0, i1, i2, i3, i4, i5, tab_hbm, out_hbm,
             idxv, gv0, gv1, ov0, ov1, semg, semo):
    wid = lax.axis_index("s") * 2 + lax.axis_index("c")
    base = wid * PER_W

    cps = [
        pltpu.async_copy(idx.at[pl.ds(base, PER_W)],
                         idxv.at[pl.ds(f * PER_W, PER_W)], semg)
        for f, idx in enumerate((i0, i1, i2, i3, i4, i5))
    ]
    for cp in cps:
        cp.wait()

    def comb_body(p, carry):
        s0 = pl.ds(0 * PER_W + p * 16, 16)
        s1 = pl.ds(1 * PER_W + p * 16, 16)
        s2 = pl.ds(2 * PER_W + p * 16, 16)
        s3 = pl.ds(3 * PER_W + p * 16, 16)
        s4 = pl.ds(4 * PER_W + p * 16, 16)
        s5 = pl.ds(5 * PER_W + p * 16, 16)
        idxv[s0] = idxv[s0] * 64 + idxv[s4]
        idxv[s1] = idxv[s1] + jnp.int32(R_BAR)
        idxv[s2] = idxv[s2] * 128 + idxv[s5] + jnp.int32(R_PD)
        idxv[s3] = idxv[s3] + jnp.int32(R_WORD)
        return carry

    lax.fori_loop(0, PER_W // 16, comb_body, 0, unroll=4)

    def fire(t, gvb):
        for g in range(NG):
            pltpu.async_copy(
                tab_hbm.at[idxv.at[pl.ds(g * PER_W + t * CH, CH)]],
                gvb.at[pl.ds(g * CH, CH)], semg)

    def drain_gathers(gvb):
        pltpu.make_async_copy(tab_hbm.at[pl.ds(0, NG * CH)], gvb, semg).wait()

    def drain_out(ovb):
        pltpu.make_async_copy(ovb, out_hbm.at[pl.ds(base, CH)], semo).wait()

    def compute(gvb, ovb):
        def tok(j, carry):
            for c in range(8):
                sl = pl.ds(c * 16, 16)
                acc = gvb[j, sl]
                for g in range(1, NG):
                    acc = acc + gvb[g * CH + j, sl]
                ovb[j, sl] = acc
            return carry
        lax.fori_loop(0, CH, tok, 0, unroll=2)

    fire(0, gv0)
    fire(1, gv1)

    def sup(s, carry):
        for half, (gvb, ovb) in enumerate(((gv0, ov0), (gv1, ov1))):
            t = 2 * s + half
            drain_gathers(gvb)

            @pl.when(s > 0)
            def _():
                drain_out(ovb)

            compute(gvb, ovb)
            pltpu.async_copy(ovb, out_hbm.at[pl.ds(base + t * CH, CH)], semo)

            @pl.when(s < NSUP - 1)
            def _():
                fire(t + 2, gvb)
        return carry

    lax.fori_loop(0, NSUP, sup, 0)
    drain_out(ov0)
    drain_out(ov1)


_sc_gather = functools.partial(
    pl.kernel,
    out_type=jax.ShapeDtypeStruct((N, D), jnp.float32),
    mesh=plsc.VectorSubcoreMesh(core_axis_name="c", subcore_axis_name="s",
                                num_cores=2),
    scratch_types=[
        pltpu.VMEM((NF * PER_W,), jnp.int32),
        pltpu.VMEM((NG * CH, D), jnp.float32),
        pltpu.VMEM((NG * CH, D), jnp.float32),
        pltpu.VMEM((CH, D), jnp.float32),
        pltpu.VMEM((CH, D), jnp.float32),
        pltpu.SemaphoreType.DMA,
        pltpu.SemaphoreType.DMA,
    ],
)(_sc_body)


@jax.jit
def kernel(tempo, global_bar, global_pos, token, vel, dur,
           W_tempo, W_bar, W_pos, W_word, W_vel, W_dur, proj_W, proj_b):
    tab = _build(W_tempo, W_bar, W_pos, W_word, W_vel, W_dur,
                 proj_W, proj_b.reshape(1, D))

    idxs = [a.astype(jnp.int32).reshape(N)
            for a in (tempo, global_bar, global_pos, token, vel, dur)]
    out = _sc_gather(*idxs, tab)
    return out.reshape(B, L, D)
